# Initial kernel scaffold; baseline (speedup 1.0000x reference)
import jax
import jax.numpy as jnp
from jax.experimental import pallas as pl


def kernel(x, edge_index, W1, b1, W2, b2, Wf1, bf1, Wf2, bf2):
    raise NotImplementedError("write your pallas kernel here")


# speedup vs baseline: 13.0783x; 13.0783x over previous
import jax
import jax.numpy as jnp
from jax import lax
from jax.experimental import pallas as pl
from jax.experimental.pallas import tpu as pltpu
from jax.experimental.pallas import tpu_sc as plsc

_NC = 2
_NS = 16
_NW = _NC * _NS
_K = 80


def _sc_mesh():
    return plsc.VectorSubcoreMesh(core_axis_name="c", subcore_axis_name="s",
                                  num_cores=_NC, num_subcores=_NS)


def _sc_deg(dst, zeros_n, ones_k, n):
    e = dst.shape[0]
    ew = e // _NW
    nch = ew // _K

    def body(dst_hbm, zeros_hbm, ones_hbm, out_hbm, acc, dst_v, ones_v):
        cid = lax.axis_index("c")
        sid = lax.axis_index("s")
        wid = cid * _NS + sid

        @pl.when(sid == 0)
        def _():
            pltpu.sync_copy(zeros_hbm, acc)

        pltpu.sync_copy(ones_hbm, ones_v)
        plsc.subcore_barrier()

        def chunk(i, carry):
            b = wid * ew + i * _K
            pltpu.sync_copy(dst_hbm.at[pl.ds(b, _K)], dst_v)
            pltpu.sync_copy(ones_v, acc.at[dst_v], add=True)
            return carry

        lax.fori_loop(0, nch, chunk, 0)
        plsc.subcore_barrier()

        @pl.when(sid == 0)
        def _():
            pltpu.sync_copy(acc, out_hbm.at[cid, 0])

    f = pl.kernel(
        body,
        out_type=jax.ShapeDtypeStruct((_NC, 1, n), jnp.float32),
        mesh=_sc_mesh(),
        scratch_types=[
            pltpu.VMEM_SHARED((n,), jnp.float32),
            pltpu.VMEM((_K,), jnp.int32),
            pltpu.VMEM((_K,), jnp.float32),
        ],
    )
    return f(dst, zeros_n, ones_k)


def _sc_edges(g, src, dst, zeros_rt):
    n, d = g.shape
    e = src.shape[0]
    ew = e // _NW
    nch = ew // _K
    rt = (n // _NS) // 8 * 8
    tail = n - rt * _NS

    def body(g_hbm, src_hbm, dst_hbm, zeros_hbm, out_hbm,
             acc, src_v, dst_v, rows_v, sem):
        cid = lax.axis_index("c")
        sid = lax.axis_index("s")
        wid = cid * _NS + sid

        pltpu.sync_copy(zeros_hbm, acc.at[pl.ds(sid * rt, rt)])

        @pl.when(sid == 0)
        def _():
            pltpu.sync_copy(zeros_hbm.at[pl.ds(0, tail)],
                            acc.at[pl.ds(rt * _NS, tail)])

        plsc.subcore_barrier()

        def chunk(i, carry):
            b = wid * ew + i * _K
            pltpu.sync_copy(src_hbm.at[pl.ds(b, _K)], src_v)
            pltpu.sync_copy(dst_hbm.at[pl.ds(b, _K)], dst_v)
            pltpu.async_copy(g_hbm.at[src_v], rows_v, sem).wait()
            pltpu.sync_copy(rows_v, acc.at[dst_v], add=True)
            return carry

        lax.fori_loop(0, nch, chunk, 0)
        plsc.subcore_barrier()
        pltpu.sync_copy(acc.at[pl.ds(sid * rt, rt)],
                        out_hbm.at[cid, pl.ds(sid * rt, rt)])

        @pl.when(sid == 0)
        def _():
            pltpu.sync_copy(acc.at[pl.ds(rt * _NS, tail)],
                            out_hbm.at[cid, pl.ds(rt * _NS, tail)])

    f = pl.kernel(
        body,
        out_type=jax.ShapeDtypeStruct((_NC, n, d), jnp.float32),
        mesh=_sc_mesh(),
        scratch_types=[
            pltpu.VMEM_SHARED((n, d), jnp.float32),
            pltpu.VMEM((_K,), jnp.int32),
            pltpu.VMEM((_K,), jnp.int32),
            pltpu.VMEM((_K, d), jnp.float32),
            pltpu.SemaphoreType.DMA,
        ],
    )
    return f(g, src, dst, zeros_rt)


def _tc_g1(degT, x, w1, bsz):
    n, d = x.shape

    def body(degT_ref, x_ref, w_ref, g_ref, dis_ref):
        deg = degT_ref[:, 0:1] + degT_ref[:, 1:2] + 1.0
        dis = lax.rsqrt(deg)
        h = jnp.dot(x_ref[...], w_ref[...], preferred_element_type=jnp.float32)
        g_ref[...] = h * dis
        dis_ref[...] = dis

    return pl.pallas_call(
        body,
        grid=(n // bsz,),
        in_specs=[
            pl.BlockSpec((bsz, 2), lambda i: (i, 0)),
            pl.BlockSpec((bsz, d), lambda i: (i, 0)),
            pl.BlockSpec((d, d), lambda i: (0, 0)),
        ],
        out_specs=[
            pl.BlockSpec((bsz, d), lambda i: (i, 0)),
            pl.BlockSpec((bsz, 1), lambda i: (i, 0)),
        ],
        out_shape=[
            jax.ShapeDtypeStruct((n, d), jnp.float32),
            jax.ShapeDtypeStruct((n, 1), jnp.float32),
        ],
    )(degT, x, w1)


def _tc_layer(sp, g, dis, b, w, bsz):
    n, d = g.shape

    def body(sp_ref, g_ref, dis_ref, b_ref, w_ref, out_ref):
        s = sp_ref[0] + sp_ref[1] + g_ref[...]
        h = jnp.tanh(dis_ref[...] * s + b_ref[...])
        out_ref[...] = dis_ref[...] * jnp.dot(
            h, w_ref[...], preferred_element_type=jnp.float32)

    return pl.pallas_call(
        body,
        grid=(n // bsz,),
        in_specs=[
            pl.BlockSpec((2, bsz, d), lambda i: (0, i, 0)),
            pl.BlockSpec((bsz, d), lambda i: (i, 0)),
            pl.BlockSpec((bsz, 1), lambda i: (i, 0)),
            pl.BlockSpec((1, d), lambda i: (0, 0)),
            pl.BlockSpec((d, d), lambda i: (0, 0)),
        ],
        out_specs=pl.BlockSpec((bsz, d), lambda i: (i, 0)),
        out_shape=jax.ShapeDtypeStruct((n, d), jnp.float32),
    )(sp, g, dis, b, w)


def _tc_head(sp, g, dis, b2, wf1, bf1, wf2, bf2, bsz):
    n, d = g.shape
    h_fc = wf1.shape[1]
    n_out = wf2.shape[1]

    def body(sp_ref, g_ref, dis_ref, b2_ref, wf1_ref, bf1_ref, wf2_ref,
             bf2_ref, out_ref):
        s = sp_ref[0] + sp_ref[1] + g_ref[...]
        h = jnp.tanh(dis_ref[...] * s + b2_ref[...])
        f = jnp.tanh(jnp.dot(h, wf1_ref[...],
                             preferred_element_type=jnp.float32) + bf1_ref[...])
        out_ref[...] = jnp.dot(
            f, wf2_ref[...], preferred_element_type=jnp.float32) + bf2_ref[...]

    return pl.pallas_call(
        body,
        grid=(n // bsz,),
        in_specs=[
            pl.BlockSpec((2, bsz, d), lambda i: (0, i, 0)),
            pl.BlockSpec((bsz, d), lambda i: (i, 0)),
            pl.BlockSpec((bsz, 1), lambda i: (i, 0)),
            pl.BlockSpec((1, d), lambda i: (0, 0)),
            pl.BlockSpec((d, h_fc), lambda i: (0, 0)),
            pl.BlockSpec((1, h_fc), lambda i: (0, 0)),
            pl.BlockSpec((h_fc, n_out), lambda i: (0, 0)),
            pl.BlockSpec((1, n_out), lambda i: (0, 0)),
        ],
        out_specs=pl.BlockSpec((bsz, n_out), lambda i: (i, 0)),
        out_shape=jax.ShapeDtypeStruct((n, n_out), jnp.float32),
    )(sp, g, dis, b2, wf1, bf1, wf2, bf2)


def kernel(x, edge_index, W1, b1, W2, b2, Wf1, bf1, Wf2, bf2):
    n, d = x.shape
    src = edge_index[0]
    dst = edge_index[1]

    zeros_n = jnp.zeros((n,), jnp.float32)
    ones_k = jnp.ones((_K,), jnp.float32)
    zeros_rt = jnp.zeros(((n // _NS) // 8 * 8, d), jnp.float32)

    bsz = 2000

    degp = _sc_deg(dst, zeros_n, ones_k, n)
    degT = degp.reshape(_NC, n).T
    g1, dis = _tc_g1(degT, x, W1, bsz)
    s1p = _sc_edges(g1, src, dst, zeros_rt)
    g2 = _tc_layer(s1p, g1, dis, b1.reshape(1, d), W2, bsz)
    s2p = _sc_edges(g2, src, dst, zeros_rt)
    out = _tc_head(s2p, g2, dis, b2.reshape(1, d), Wf1,
                   bf1.reshape(1, -1), Wf2, bf2.reshape(1, -1), bsz)
    return out


# speedup vs baseline: 33.7728x; 2.5824x over previous
import jax
import jax.numpy as jnp
from jax import lax
from jax.experimental import pallas as pl
from jax.experimental.pallas import tpu as pltpu
from jax.experimental.pallas import tpu_sc as plsc

_NC = 2
_NS = 16
_NW = _NC * _NS
_K = 125
_NBUF = 2
_IR = 4


def _sc_mesh():
    return plsc.VectorSubcoreMesh(core_axis_name="c", subcore_axis_name="s",
                                  num_cores=_NC, num_subcores=_NS)


def _sc_deg(idx_rs, zeros_n, ones_k, n):
    nch = idx_rs.shape[1]

    def body(idx_hbm, zeros_hbm, ones_hbm, out_hbm, acc, idx_all, ones_v):
        cid = lax.axis_index("c")
        sid = lax.axis_index("s")
        wid = cid * _NS + sid

        @pl.when(sid == 0)
        def _():
            pltpu.sync_copy(zeros_hbm, acc)

        pltpu.sync_copy(idx_hbm.at[wid], idx_all)
        pltpu.sync_copy(ones_hbm, ones_v)
        plsc.subcore_barrier()

        def chunk(i, carry):
            pltpu.sync_copy(ones_v, acc.at[idx_all.at[i, 1]], add=True)
            return carry

        lax.fori_loop(0, nch, chunk, 0)
        plsc.subcore_barrier()

        @pl.when(sid == 0)
        def _():
            pltpu.sync_copy(acc, out_hbm.at[cid, 0])

    f = pl.kernel(
        body,
        out_type=jax.ShapeDtypeStruct((_NC, 1, n), jnp.float32),
        mesh=_sc_mesh(),
        scratch_types=[
            pltpu.VMEM_SHARED((n,), jnp.float32),
            pltpu.VMEM((nch, 2, _K), jnp.int32),
            pltpu.VMEM((_K,), jnp.float32),
        ],
    )
    return f(idx_rs, zeros_n, ones_k)


def _sc_edges(g, idx_rs, zeros_rt):
    n, d = g.shape
    nch = idx_rs.shape[1]
    rt = (n // _NS) // 8 * 8
    tail = n - rt * _NS

    def body(g_hbm, idx_hbm, zeros_hbm, out_hbm, acc, idxb, rows, gsem, isem):
        cid = lax.axis_index("c")
        sid = lax.axis_index("s")
        wid = cid * _NS + sid

        def idx_start(ch, slot):
            pltpu.async_copy(idx_hbm.at[wid, ch], idxb.at[slot],
                             isem.at[slot])

        def idx_wait(slot):
            pltpu.make_async_copy(idx_hbm.at[wid, 0], idxb.at[slot],
                                  isem.at[slot]).wait()

        def gather_start(islot, rslot):
            pltpu.async_copy(g_hbm.at[idxb.at[islot, 0]], rows.at[rslot],
                             gsem.at[rslot])

        def gather_wait(rslot):
            pltpu.make_async_copy(g_hbm.at[idxb.at[0, 0]], rows.at[rslot],
                                  gsem.at[rslot]).wait()

        pltpu.sync_copy(zeros_hbm, acc.at[pl.ds(sid * rt, rt)])

        @pl.when(sid == 0)
        def _():
            pltpu.sync_copy(zeros_hbm.at[pl.ds(0, tail)],
                            acc.at[pl.ds(rt * _NS, tail)])

        plsc.subcore_barrier()

        for r in range(_IR):
            idx_start(r, r)
        for b in range(_NBUF):
            idx_wait(b)
            gather_start(b, b)

        def outer(gi, carry):
            for b in range(_IR):
                i = gi * _IR + b
                gather_wait(b % _NBUF)
                pltpu.sync_copy(rows.at[b % _NBUF], acc.at[idxb.at[b, 1]],
                                add=True)
                idx_start(i + _IR, b)
                idx_wait((b + _NBUF) % _IR)
                gather_start((b + _NBUF) % _IR, b % _NBUF)
            return carry

        lax.fori_loop(0, nch // _IR - 1, outer, 0)

        for b in range(_IR):
            i = nch - _IR + b
            gather_wait(b % _NBUF)
            pltpu.sync_copy(rows.at[b % _NBUF], acc.at[idxb.at[b, 1]],
                            add=True)
            if b < _IR - _NBUF:
                idx_wait((b + _NBUF) % _IR)
                gather_start((b + _NBUF) % _IR, b % _NBUF)

        plsc.subcore_barrier()
        pltpu.sync_copy(acc.at[pl.ds(sid * rt, rt)],
                        out_hbm.at[cid, pl.ds(sid * rt, rt)])

        @pl.when(sid == 0)
        def _():
            pltpu.sync_copy(acc.at[pl.ds(rt * _NS, tail)],
                            out_hbm.at[cid, pl.ds(rt * _NS, tail)])

    f = pl.kernel(
        body,
        out_type=jax.ShapeDtypeStruct((_NC, n, d), jnp.float32),
        mesh=_sc_mesh(),
        scratch_types=[
            pltpu.VMEM_SHARED((n, d), jnp.float32),
            pltpu.VMEM((_IR, 2, _K), jnp.int32),
            pltpu.VMEM((_NBUF, _K, d), jnp.float32),
            pltpu.SemaphoreType.DMA((_NBUF,)),
            pltpu.SemaphoreType.DMA((_IR,)),
        ],
    )
    return f(g, idx_rs, zeros_rt)


def _tc_g1(degT, x, w1, bsz):
    n, d = x.shape

    def body(degT_ref, x_ref, w_ref, g_ref, dis_ref):
        deg = degT_ref[:, 0:1] + degT_ref[:, 1:2] + 1.0
        dis = lax.rsqrt(deg)
        h = jnp.dot(x_ref[...], w_ref[...], preferred_element_type=jnp.float32)
        g_ref[...] = h * dis
        dis_ref[...] = dis

    return pl.pallas_call(
        body,
        grid=(n // bsz,),
        in_specs=[
            pl.BlockSpec((bsz, 2), lambda i: (i, 0)),
            pl.BlockSpec((bsz, d), lambda i: (i, 0)),
            pl.BlockSpec((d, d), lambda i: (0, 0)),
        ],
        out_specs=[
            pl.BlockSpec((bsz, d), lambda i: (i, 0)),
            pl.BlockSpec((bsz, 1), lambda i: (i, 0)),
        ],
        out_shape=[
            jax.ShapeDtypeStruct((n, d), jnp.float32),
            jax.ShapeDtypeStruct((n, 1), jnp.float32),
        ],
    )(degT, x, w1)


def _tc_layer(sp, g, dis, b, w, bsz):
    n, d = g.shape

    def body(sp_ref, g_ref, dis_ref, b_ref, w_ref, out_ref):
        s = sp_ref[0] + sp_ref[1] + g_ref[...]
        h = jnp.tanh(dis_ref[...] * s + b_ref[...])
        out_ref[...] = dis_ref[...] * jnp.dot(
            h, w_ref[...], preferred_element_type=jnp.float32)

    return pl.pallas_call(
        body,
        grid=(n // bsz,),
        in_specs=[
            pl.BlockSpec((2, bsz, d), lambda i: (0, i, 0)),
            pl.BlockSpec((bsz, d), lambda i: (i, 0)),
            pl.BlockSpec((bsz, 1), lambda i: (i, 0)),
            pl.BlockSpec((1, d), lambda i: (0, 0)),
            pl.BlockSpec((d, d), lambda i: (0, 0)),
        ],
        out_specs=pl.BlockSpec((bsz, d), lambda i: (i, 0)),
        out_shape=jax.ShapeDtypeStruct((n, d), jnp.float32),
    )(sp, g, dis, b, w)


def _tc_head(sp, g, dis, b2, wf1, bf1, wf2, bf2, bsz):
    n, d = g.shape
    h_fc = wf1.shape[1]
    n_out = wf2.shape[1]

    def body(sp_ref, g_ref, dis_ref, b2_ref, wf1_ref, bf1_ref, wf2_ref,
             bf2_ref, out_ref):
        s = sp_ref[0] + sp_ref[1] + g_ref[...]
        h = jnp.tanh(dis_ref[...] * s + b2_ref[...])
        f = jnp.tanh(jnp.dot(h, wf1_ref[...],
                             preferred_element_type=jnp.float32) + bf1_ref[...])
        out_ref[...] = jnp.dot(
            f, wf2_ref[...], preferred_element_type=jnp.float32) + bf2_ref[...]

    return pl.pallas_call(
        body,
        grid=(n // bsz,),
        in_specs=[
            pl.BlockSpec((2, bsz, d), lambda i: (0, i, 0)),
            pl.BlockSpec((bsz, d), lambda i: (i, 0)),
            pl.BlockSpec((bsz, 1), lambda i: (i, 0)),
            pl.BlockSpec((1, d), lambda i: (0, 0)),
            pl.BlockSpec((d, h_fc), lambda i: (0, 0)),
            pl.BlockSpec((1, h_fc), lambda i: (0, 0)),
            pl.BlockSpec((h_fc, n_out), lambda i: (0, 0)),
            pl.BlockSpec((1, n_out), lambda i: (0, 0)),
        ],
        out_specs=pl.BlockSpec((bsz, n_out), lambda i: (i, 0)),
        out_shape=jax.ShapeDtypeStruct((n, n_out), jnp.float32),
    )(sp, g, dis, b2, wf1, bf1, wf2, bf2)


def kernel(x, edge_index, W1, b1, W2, b2, Wf1, bf1, Wf2, bf2):
    n, d = x.shape
    e = edge_index.shape[1]
    ew = e // _NW
    nch = ew // _K
    idx_rs = edge_index.reshape(2, _NW, nch, _K).transpose(1, 2, 0, 3)

    zeros_n = jnp.zeros((n,), jnp.float32)
    ones_k = jnp.ones((_K,), jnp.float32)
    zeros_rt = jnp.zeros(((n // _NS) // 8 * 8, d), jnp.float32)

    bsz = 2000

    degp = _sc_deg(idx_rs, zeros_n, ones_k, n)
    degT = degp.reshape(_NC, n).T
    g1, dis = _tc_g1(degT, x, W1, bsz)
    s1p = _sc_edges(g1, idx_rs, zeros_rt)
    g2 = _tc_layer(s1p, g1, dis, b1.reshape(1, d), W2, bsz)
    s2p = _sc_edges(g2, idx_rs, zeros_rt)
    out = _tc_head(s2p, g2, dis, b2.reshape(1, d), Wf1,
                   bf1.reshape(1, -1), Wf2, bf2.reshape(1, -1), bsz)
    return out
